# baseline (speedup 1.0000x reference)
import jax
import jax.numpy as jnp
from jax import lax
from jax.experimental import pallas as pl
from jax.experimental.pallas import tpu as pltpu
from jax.experimental.pallas import tpu_sc as plsc

L = 16
NW = 32
N = 21824
NG = 200
GP = 208
B = 224
PER_TILE = 688
SENT = 2.0e9
NGRP = 12

CHUNKS = ((0, 512, 512, 0), (16384, 128, 128, 512),
          (20480, 32, 32, 640), (21504, 16, 16, 672))


def _sc_body(ax_h, ay_h, bx_h, by_h, gts_h, out_h,
             av, gv, bcx, bcy, bgi, outv,
             sem_g, sem_a0, sem_a1, sem_a2, sem_a3):
    nc = 2
    wid = lax.axis_index("s") * nc + lax.axis_index("c")
    asems = (sem_a0, sem_a1, sem_a2, sem_a3)
    comps = (ax_h, ay_h, bx_h, by_h)

    cp_g = pltpu.async_copy(gts_h, gv, sem_g)
    copies = []
    for (base, step, cnt, voff), sem in zip(CHUNKS[:3], asems[:3]):
        start = pl.multiple_of(base + step * wid, 16)
        for cc in range(4):
            copies.append(pltpu.async_copy(
                comps[cc].at[pl.ds(start, cnt)],
                av.at[pl.ds(688 * cc + voff, cnt)], sem))
    small = wid < 20

    @pl.when(small)
    def _():
        start = pl.multiple_of(21504 + 16 * wid, 16)
        hs = [pltpu.async_copy(comps[cc].at[pl.ds(start, 16)],
                               av.at[pl.ds(688 * cc + 672, 16)], sem_a3)
              for cc in range(4)]
        for h in hs:
            h.wait()

    sent_vec = jnp.full((L,), SENT, jnp.float32)
    iota = lax.iota(jnp.int32, L)
    cp_g.wait()

    def prep(k, cnts):
        o = k * L
        gidx = iota + o
        valid = gidx < NG
        rows = jnp.where(valid, gidx, 0)
        x0 = plsc.load_gather(gv, [rows, jnp.full((L,), 0, jnp.int32)])
        y0 = plsc.load_gather(gv, [rows, jnp.full((L,), 1, jnp.int32)])
        x1 = plsc.load_gather(gv, [rows, jnp.full((L,), 2, jnp.int32)])
        y1 = plsc.load_gather(gv, [rows, jnp.full((L,), 3, jnp.int32)])
        area = (x1 - x0) * (y1 - y0)
        lv = jnp.zeros((L,), jnp.float32)
        for thr in (1024.0, 4096.0, 16384.0, 65536.0):
            lv = lv + jnp.where(area >= thr, 1.0, 0.0).astype(jnp.float32)
        lv = jnp.where(area >= 262144.0, 0.0, lv)
        cx = (x0 + x1) * 0.5
        cy = (y0 + y1) * 0.5
        out = []
        for l in range(5):
            m = (lv == float(l)) & valid
            r = plsc.cumsum(m.astype(jnp.int32))
            dest = cnts[l] + r + (B * l - 1)
            plsc.store_scatter(bcx, [dest], cx, mask=m)
            plsc.store_scatter(bcy, [dest], cy, mask=m)
            plsc.store_scatter(bgi, [dest], gidx, mask=m)
            out.append(cnts[l] + plsc.all_reduce_population_count(m))
        return tuple(out)

    cnts = lax.fori_loop(0, GP // L, prep,
                         tuple(jnp.zeros((L,), jnp.int32) for _ in range(5)))
    c = [jnp.max(cnts[l]) for l in range(5)]
    for l in range(5):
        plsc.store_scatter(bcx, [iota + (B * l + c[l])], sent_vec)

    for handle in copies:
        handle.wait()

    neg2 = jnp.full((L,), -2, jnp.int32)

    def scan_streams(starts, nq, boxes):
        @plsc.parallel_loop(0, nq, carry=(neg2, neg2, neg2, neg2))
        def scan(q, assigns):
            out = list(assigns)
            for u in range(4):
                idx = jnp.full((L,), starts[u] + q, jnp.int32)
                bxv = plsc.load_gather(bcx, [idx])
                byv = plsc.load_gather(bcy, [idx])
                bgv = plsc.load_gather(bgi, [idx])
                a0, a1, a2, a3 = boxes[u]
                m = (bxv > a0) & (byv > a1) & (bxv < a2) & (byv < a3)
                out[u] = jnp.where(m, bgv, out[u])
            return tuple(out)

        return scan

    def load_box(row):
        return [plsc.load_gather(av, [row + 688 * cc]) for cc in range(4)]

    def group(g, carry):
        rows = [iota + (64 * g + 16 * u) for u in range(4)]
        bb = jnp.where(g < 8, 0, B)
        n = jnp.where(g < 8, c[0], c[1])

        @pl.when(n == 0)
        def _():
            for u in range(4):
                plsc.store_scatter(outv, [rows[u]], neg2)

        @pl.when(n > 0)
        def _():
            boxes = [load_box(rows[u]) for u in range(4)]
            nq = (n + 3) >> 2

            @plsc.parallel_loop(0, nq, carry=(neg2, neg2, neg2, neg2))
            def scan(q, assigns):
                out = list(assigns)
                j = bb + q * 4
                for u in range(4):
                    idx = jnp.full((L,), j + u, jnp.int32)
                    bxv = plsc.load_gather(bcx, [idx])
                    byv = plsc.load_gather(bcy, [idx])
                    bgv = plsc.load_gather(bgi, [idx])
                    for si, (a0, a1, a2, a3) in enumerate(boxes):
                        m = (bxv > a0) & (byv > a1) & (bxv < a2) & (byv < a3)
                        out[si] = jnp.where(m, bgv, out[si])
                return tuple(out)

            for u in range(4):
                plsc.store_scatter(outv, [rows[u]], scan[u])
        return carry

    lax.fori_loop(0, 10, group, 0)

    o0 = pltpu.async_copy(outv.at[pl.ds(0, 512)],
                          out_h.at[pl.ds(pl.multiple_of(512 * wid, 16), 512)],
                          sem_a0)
    o1 = pltpu.async_copy(outv.at[pl.ds(512, 128)],
                          out_h.at[pl.ds(pl.multiple_of(16384 + 128 * wid, 16), 128)],
                          sem_a1)

    r40 = iota + 640
    r41 = iota + 656
    h2 = (c[2] + 1) >> 1
    b40 = load_box(r40)
    b41 = load_box(r41)
    p = scan_streams([2 * B, 2 * B, 2 * B + h2, 2 * B + h2], h2,
                     [b40, b41, b40, b41])
    plsc.store_scatter(outv, [r40], jnp.maximum(p[0], p[2]))
    plsc.store_scatter(outv, [r41], jnp.maximum(p[1], p[3]))

    r42 = iota + 672
    bb42 = jnp.where(wid < 16, 3 * B, 4 * B)
    n42 = jnp.where(small, jnp.where(wid < 16, c[3], c[4]), 0)
    q4 = (n42 + 3) >> 2
    b42 = load_box(r42)
    p = scan_streams([bb42, bb42 + q4, bb42 + 2 * q4, bb42 + 3 * q4], q4,
                     [b42, b42, b42, b42])
    plsc.store_scatter(outv, [r42],
                       jnp.maximum(jnp.maximum(p[0], p[1]),
                                   jnp.maximum(p[2], p[3])))

    o2 = pltpu.async_copy(outv.at[pl.ds(640, 32)],
                          out_h.at[pl.ds(pl.multiple_of(20480 + 32 * wid, 16), 32)],
                          sem_a2)

    @pl.when(small)
    def _():
        pltpu.async_copy(outv.at[pl.ds(672, 16)],
                         out_h.at[pl.ds(pl.multiple_of(21504 + 16 * wid, 16), 16)],
                         sem_a3).wait()

    o0.wait()
    o1.wait()
    o2.wait()


@jax.jit
def kernel(anchor, gts):
    cols = [anchor[:, cc] for cc in range(4)]

    mesh = plsc.VectorSubcoreMesh(core_axis_name="c", subcore_axis_name="s")
    run = pl.kernel(
        _sc_body,
        mesh=mesh,
        compiler_params=pltpu.CompilerParams(needs_layout_passes=False,
                                             skip_device_barrier=True),
        out_type=jax.ShapeDtypeStruct((N,), jnp.int32),
        scratch_types=[
            pltpu.VMEM((4 * PER_TILE,), jnp.float32),
            pltpu.VMEM((NG, 4), jnp.float32),
            pltpu.VMEM((5 * B,), jnp.float32),
            pltpu.VMEM((5 * B,), jnp.float32),
            pltpu.VMEM((5 * B,), jnp.int32),
            pltpu.VMEM((PER_TILE,), jnp.int32),
            pltpu.SemaphoreType.DMA,
            pltpu.SemaphoreType.DMA,
            pltpu.SemaphoreType.DMA,
            pltpu.SemaphoreType.DMA,
            pltpu.SemaphoreType.DMA,
        ],
    )
    return run(*cols, gts).astype(jnp.int64)


# speedup vs baseline: 1.1113x; 1.1113x over previous
import jax
import jax.numpy as jnp
from jax import lax
from jax.experimental import pallas as pl
from jax.experimental.pallas import tpu as pltpu
from jax.experimental.pallas import tpu_sc as plsc

L = 16
NW = 32
N = 21824
NG = 200
GP = 208
B = 224
PER_TILE = 688
SENT = 2.0e9
NGRP = 12

CHUNKS = ((0, 512, 512, 0), (16384, 128, 128, 512),
          (20480, 32, 32, 640), (21504, 16, 16, 672))


def _sc_body(ax_h, ay_h, bx_h, by_h, gts_h, out_h,
             av, gv, bcx, bcy, bgi, outv,
             sem_g, sem_a0, sem_a1, sem_a2, sem_a3):
    nc = 2
    wid = lax.axis_index("s") * nc + lax.axis_index("c")
    asems = (sem_a0, sem_a1, sem_a2, sem_a3)
    comps = (ax_h, ay_h, bx_h, by_h)

    cp_g = pltpu.async_copy(gts_h, gv, sem_g)
    copies = []
    for (base, step, cnt, voff), sem in zip(CHUNKS[:3], asems[:3]):
        start = pl.multiple_of(base + step * wid, 16)
        for cc in range(4):
            copies.append(pltpu.async_copy(
                comps[cc].at[pl.ds(start, cnt)],
                av.at[pl.ds(688 * cc + voff, cnt)], sem))
    small = wid < 20

    @pl.when(small)
    def _():
        start = pl.multiple_of(21504 + 16 * wid, 16)
        hs = [pltpu.async_copy(comps[cc].at[pl.ds(start, 16)],
                               av.at[pl.ds(688 * cc + 672, 16)], sem_a3)
              for cc in range(4)]
        for h in hs:
            h.wait()

    sent_vec = jnp.full((L,), SENT, jnp.float32)
    iota = lax.iota(jnp.int32, L)
    cp_g.wait()

    def prep(k, cnts):
        o = k * L
        x0 = gv[pl.ds(0 * GP + o, L)]
        y0 = gv[pl.ds(1 * GP + o, L)]
        x1 = gv[pl.ds(2 * GP + o, L)]
        y1 = gv[pl.ds(3 * GP + o, L)]
        area = (x1 - x0) * (y1 - y0)
        lv = jnp.zeros((L,), jnp.float32)
        for thr in (1024.0, 4096.0, 16384.0, 65536.0):
            lv = lv + jnp.where(area >= thr, 1.0, 0.0).astype(jnp.float32)
        lv = jnp.where(area >= 262144.0, 0.0, lv)
        cx = (x0 + x1) * 0.5
        cy = (y0 + y1) * 0.5
        gidx = iota + o
        valid = gidx < NG
        out = []
        for l in range(5):
            m = (lv == float(l)) & valid
            r = plsc.cumsum(m.astype(jnp.int32))
            dest = cnts[l] + r + (B * l - 1)
            plsc.store_scatter(bcx, [dest], cx, mask=m)
            plsc.store_scatter(bcy, [dest], cy, mask=m)
            plsc.store_scatter(bgi, [dest], gidx, mask=m)
            out.append(cnts[l] + plsc.all_reduce_population_count(m))
        return tuple(out)

    cnts = lax.fori_loop(0, GP // L, prep,
                         tuple(jnp.zeros((L,), jnp.int32) for _ in range(5)))
    c = [jnp.max(cnts[l]) for l in range(5)]
    for l in range(5):
        plsc.store_scatter(bcx, [iota + (B * l + c[l])], sent_vec)

    for handle in copies:
        handle.wait()

    neg2 = jnp.full((L,), -2, jnp.int32)

    def scan_streams(starts, nq, boxes):
        @plsc.parallel_loop(0, nq, carry=(neg2, neg2, neg2, neg2))
        def scan(q, assigns):
            out = list(assigns)
            for u in range(4):
                idx = jnp.full((L,), starts[u] + q, jnp.int32)
                bxv = plsc.load_gather(bcx, [idx])
                byv = plsc.load_gather(bcy, [idx])
                bgv = plsc.load_gather(bgi, [idx])
                a0, a1, a2, a3 = boxes[u]
                m = (bxv > a0) & (byv > a1) & (bxv < a2) & (byv < a3)
                out[u] = jnp.where(m, bgv, out[u])
            return tuple(out)

        return scan

    def load_box(row):
        return [plsc.load_gather(av, [row + 688 * cc]) for cc in range(4)]

    def group(g, carry):
        rows = [iota + (64 * g + 16 * u) for u in range(4)]
        bb = jnp.where(g < 8, 0, B)
        n = jnp.where(g < 8, c[0], c[1])

        @pl.when(n == 0)
        def _():
            for u in range(4):
                plsc.store_scatter(outv, [rows[u]], neg2)

        @pl.when(n > 0)
        def _():
            boxes = [load_box(rows[u]) for u in range(4)]
            nq = (n + 3) >> 2

            @plsc.parallel_loop(0, nq, carry=(neg2, neg2, neg2, neg2))
            def scan(q, assigns):
                out = list(assigns)
                j = bb + q * 4
                for u in range(4):
                    idx = jnp.full((L,), j + u, jnp.int32)
                    bxv = plsc.load_gather(bcx, [idx])
                    byv = plsc.load_gather(bcy, [idx])
                    bgv = plsc.load_gather(bgi, [idx])
                    for si, (a0, a1, a2, a3) in enumerate(boxes):
                        m = (bxv > a0) & (byv > a1) & (bxv < a2) & (byv < a3)
                        out[si] = jnp.where(m, bgv, out[si])
                return tuple(out)

            for u in range(4):
                plsc.store_scatter(outv, [rows[u]], scan[u])
        return carry

    lax.fori_loop(0, 10, group, 0)

    o0 = pltpu.async_copy(outv.at[pl.ds(0, 512)],
                          out_h.at[pl.ds(pl.multiple_of(512 * wid, 16), 512)],
                          sem_a0)
    o1 = pltpu.async_copy(outv.at[pl.ds(512, 128)],
                          out_h.at[pl.ds(pl.multiple_of(16384 + 128 * wid, 16), 128)],
                          sem_a1)

    r40 = iota + 640
    r41 = iota + 656
    h2 = (c[2] + 1) >> 1
    b40 = load_box(r40)
    b41 = load_box(r41)
    p = scan_streams([2 * B, 2 * B, 2 * B + h2, 2 * B + h2], h2,
                     [b40, b41, b40, b41])
    plsc.store_scatter(outv, [r40], jnp.maximum(p[0], p[2]))
    plsc.store_scatter(outv, [r41], jnp.maximum(p[1], p[3]))

    r42 = iota + 672
    bb42 = jnp.where(wid < 16, 3 * B, 4 * B)
    n42 = jnp.where(small, jnp.where(wid < 16, c[3], c[4]), 0)
    q4 = (n42 + 3) >> 2
    b42 = load_box(r42)
    p = scan_streams([bb42, bb42 + q4, bb42 + 2 * q4, bb42 + 3 * q4], q4,
                     [b42, b42, b42, b42])
    plsc.store_scatter(outv, [r42],
                       jnp.maximum(jnp.maximum(p[0], p[1]),
                                   jnp.maximum(p[2], p[3])))

    o2 = pltpu.async_copy(outv.at[pl.ds(640, 32)],
                          out_h.at[pl.ds(pl.multiple_of(20480 + 32 * wid, 16), 32)],
                          sem_a2)

    @pl.when(small)
    def _():
        pltpu.async_copy(outv.at[pl.ds(672, 16)],
                         out_h.at[pl.ds(pl.multiple_of(21504 + 16 * wid, 16), 16)],
                         sem_a3).wait()

    o0.wait()
    o1.wait()
    o2.wait()


@jax.jit
def kernel(anchor, gts):
    cols = [anchor[:, cc] for cc in range(4)]
    gflat = jnp.pad(gts.T, ((0, 0), (0, GP - NG)),
                    constant_values=SENT).reshape(-1)

    mesh = plsc.VectorSubcoreMesh(core_axis_name="c", subcore_axis_name="s")
    run = pl.kernel(
        _sc_body,
        mesh=mesh,
        compiler_params=pltpu.CompilerParams(needs_layout_passes=False,
                                             skip_device_barrier=True),
        out_type=jax.ShapeDtypeStruct((N,), jnp.int32),
        scratch_types=[
            pltpu.VMEM((4 * PER_TILE,), jnp.float32),
            pltpu.VMEM((4 * GP,), jnp.float32),
            pltpu.VMEM((5 * B,), jnp.float32),
            pltpu.VMEM((5 * B,), jnp.float32),
            pltpu.VMEM((5 * B,), jnp.int32),
            pltpu.VMEM((PER_TILE,), jnp.int32),
            pltpu.SemaphoreType.DMA,
            pltpu.SemaphoreType.DMA,
            pltpu.SemaphoreType.DMA,
            pltpu.SemaphoreType.DMA,
            pltpu.SemaphoreType.DMA,
        ],
    )
    return run(*cols, gflat).astype(jnp.int64)
